# baseline (speedup 1.0000x reference)
import jax
import jax.numpy as jnp
from jax import lax
from jax.experimental import pallas as pl
from jax.experimental.pallas import tpu as pltpu
from jax.experimental.pallas import tpu_sc as plsc

NC = 2
NS = 16
L = 16

N_TRIPLES = 4 * 3 * 512 * 512 // 3
N_FLAT = N_TRIPLES * 3

NW = NC * NS
T_PER_W = N_TRIPLES // NW
SUB = 2048
N_SUB = T_PER_W // SUB
IDX_CHUNK = 128
N_G = SUB // IDX_CHUNK


def _tanh(t):
    e = jnp.exp(t + t)
    return 1.0 - 2.0 / (e + 1.0)


def _sc_body(x_hbm, wk_hbm, out_hbm, x_v, idx_v, g_v, out_v, sem):
    wid = lax.axis_index("s") * NC + lax.axis_index("c")
    iota = lax.iota(jnp.int32, L)

    def sub_chunk(s, carry):
        fbase = (wid * T_PER_W + s * SUB) * 3
        pltpu.sync_copy(x_hbm.at[pl.ds(fbase, SUB * 3)], x_v)

        def body_a(j, c):
            p = j * (3 * L) + iota * 3
            f0 = plsc.load_gather(x_v, [p])
            f1 = plsc.load_gather(x_v, [p + 1])
            f2 = plsc.load_gather(x_v, [p + 2])
            ind = f0 * 65536.0 + f1 * 256.0 + f2
            idx_v[pl.ds(j * L, L)] = ind.astype(jnp.int32)
            return c

        lax.fori_loop(0, SUB // L, body_a, None)

        copies = []
        for g in range(N_G):
            isl = idx_v.at[pl.ds(g * IDX_CHUNK, IDX_CHUNK)]
            copies.append(pltpu.async_copy(
                wk_hbm.at[isl], g_v.at[pl.ds(g * IDX_CHUNK, IDX_CHUNK)], sem))
        for c in copies:
            c.wait()

        def body_b(j, c):
            r = j * L + iota
            r3 = j * (3 * L) + iota * 3
            for ch in range(3):
                xc = plsc.load_gather(x_v, [r3 + ch])
                wc = plsc.load_gather(g_v, [r, jnp.full((L,), ch, jnp.int32)])
                kc = plsc.load_gather(g_v, [r, jnp.full((L,), ch + 3, jnp.int32)])
                plsc.store_scatter(out_v, [r3 + ch], _tanh(xc * wc + kc))
            return c

        lax.fori_loop(0, SUB // L, body_b, None)
        pltpu.sync_copy(out_v, out_hbm.at[pl.ds(fbase, SUB * 3)])
        return carry

    lax.fori_loop(0, N_SUB, sub_chunk, None)


@jax.jit
def _colormap_sc(xf, wk):
    kern = pl.kernel(
        _sc_body,
        out_type=jax.ShapeDtypeStruct((N_FLAT,), jnp.float32),
        mesh=plsc.VectorSubcoreMesh(core_axis_name="c", subcore_axis_name="s"),
        scratch_types=[
            pltpu.VMEM((SUB * 3,), jnp.float32),
            pltpu.VMEM((SUB,), jnp.int32),
            pltpu.VMEM((SUB, 8), jnp.float32),
            pltpu.VMEM((SUB * 3,), jnp.float32),
            pltpu.SemaphoreType.DMA,
        ],
        compiler_params=pltpu.CompilerParams(
            needs_layout_passes=False, use_tc_tiling_on_sc=False),
    )
    return kern(xf, wk)


def kernel(x, w, k):
    b, c, h, wd = x.shape
    pad = jnp.zeros((w.shape[0], 2), jnp.float32)
    wk = jnp.concatenate([w, k, pad], axis=1)
    out = _colormap_sc(x.reshape(-1), wk)
    return out.reshape(-1, 3, h, wd)


# speedup vs baseline: 7.4497x; 7.4497x over previous
import jax
import jax.numpy as jnp
from jax import lax
from jax.experimental import pallas as pl
from jax.experimental.pallas import tpu as pltpu
from jax.experimental.pallas import tpu_sc as plsc

NC = 2
NS = 16
L = 16

N_TRIPLES = 4 * 3 * 512 * 512 // 3
N_FLAT = N_TRIPLES * 3

NW = NC * NS
T_PER_W = N_TRIPLES // NW
SUB = 1024
N_SUB = T_PER_W // SUB
IDX_CHUNK = 128
N_G = SUB // IDX_CHUNK


def _tanh(t):
    e = jnp.exp(t + t)
    return 1.0 - 2.0 / (e + 1.0)


def _sc_body(x_hbm, w0_hbm, w1_hbm, w2_hbm, k0_hbm, k1_hbm, k2_hbm, out_hbm,
             x0_v, x1_v, i0_v, i1_v, g0_v, g1_v, out_v, sg0, sg1):
    wid = lax.axis_index("s") * NC + lax.axis_index("c")
    iota = lax.iota(jnp.int32, L)
    tabs = (w0_hbm, w1_hbm, w2_hbm, k0_hbm, k1_hbm, k2_hbm)
    x_v = (x0_v, x1_v)
    idx_v = (i0_v, i1_v)
    g_v = (g0_v, g1_v)
    sems = (sg0, sg1)

    def fbase(s):
        return (wid * T_PER_W + s * SUB) * 3

    def pass_a(s, h):
        pltpu.sync_copy(x_hbm.at[pl.ds(fbase(s), SUB * 3)], x_v[h])

        def body_a(j, c):
            p = j * (3 * L) + iota * 3
            f0 = plsc.load_gather(x_v[h], [p])
            f1 = plsc.load_gather(x_v[h], [p + 1])
            f2 = plsc.load_gather(x_v[h], [p + 2])
            ind = f0 * 65536.0 + f1 * 256.0 + f2
            idx_v[h][pl.ds(j * L, L)] = ind.astype(jnp.int32)
            return c

        lax.fori_loop(0, SUB // L, body_a, None)

    def fire(h):
        for g in range(N_G):
            isl = idx_v[h].at[pl.ds(g * IDX_CHUNK, IDX_CHUNK)]
            for t, tab in enumerate(tabs):
                dsl = pl.ds(t * SUB + g * IDX_CHUNK, IDX_CHUNK)
                pltpu.async_copy(tab.at[isl], g_v[h].at[dsl], sems[h])

    def drain(h):
        pltpu.make_async_copy(
            x_hbm.at[pl.ds(0, SUB * 6)], g_v[h], sems[h]).wait()

    def pass_b(s, h):
        def body_b(j, c):
            r3 = j * (3 * L) + iota * 3
            sl = pl.ds(j * L, L)
            for ch in range(3):
                xc = plsc.load_gather(x_v[h], [r3 + ch])
                wc = g_v[h][pl.ds(ch * SUB + j * L, L)]
                kc = g_v[h][pl.ds((3 + ch) * SUB + j * L, L)]
                plsc.store_scatter(out_v, [r3 + ch], _tanh(xc * wc + kc))
            return c

        lax.fori_loop(0, SUB // L, body_b, None)
        pltpu.sync_copy(out_v, out_hbm.at[pl.ds(fbase(s), SUB * 3)])

    pass_a(0, 0)
    fire(0)

    def pair(j, carry):
        s1 = 2 * j + 1
        pass_a(s1, 1)
        fire(1)
        drain(0)
        pass_b(s1 - 1, 0)
        s2 = 2 * j + 2
        pass_a(s2, 0)
        fire(0)
        drain(1)
        pass_b(s1, 1)
        return carry

    lax.fori_loop(0, N_SUB // 2 - 1, pair, None)
    s_last = N_SUB - 1
    pass_a(s_last, 1)
    fire(1)
    drain(0)
    pass_b(s_last - 1, 0)
    drain(1)
    pass_b(s_last, 1)


@jax.jit
def _colormap_sc(xf, w0, w1, w2, k0, k1, k2):
    kern = pl.kernel(
        _sc_body,
        out_type=jax.ShapeDtypeStruct((N_FLAT,), jnp.float32),
        mesh=plsc.VectorSubcoreMesh(core_axis_name="c", subcore_axis_name="s"),
        scratch_types=[
            pltpu.VMEM((SUB * 3,), jnp.float32),
            pltpu.VMEM((SUB * 3,), jnp.float32),
            pltpu.VMEM((SUB,), jnp.int32),
            pltpu.VMEM((SUB,), jnp.int32),
            pltpu.VMEM((SUB * 6,), jnp.float32),
            pltpu.VMEM((SUB * 6,), jnp.float32),
            pltpu.VMEM((SUB * 3,), jnp.float32),
            pltpu.SemaphoreType.DMA,
            pltpu.SemaphoreType.DMA,
        ],
        compiler_params=pltpu.CompilerParams(
            needs_layout_passes=False, use_tc_tiling_on_sc=False),
    )
    return kern(xf, w0, w1, w2, k0, k1, k2)


def kernel(x, w, k):
    b, c, h, wd = x.shape
    out = _colormap_sc(x.reshape(-1),
                       w[:, 0], w[:, 1], w[:, 2],
                       k[:, 0], k[:, 1], k[:, 2])
    return out.reshape(-1, 3, h, wd)
